# baseline (speedup 1.0000x reference)
import functools

import jax
import jax.numpy as jnp
from jax import lax
from jax.experimental import pallas as pl
from jax.experimental.pallas import tpu as pltpu
from jax.experimental.pallas import tpu_sc as plsc

N = 10000
NP = 10240
E = 160000
G = 512
HID = 256
HHALF = 128
MLP_IN = 128
OUT = 128

NC = 2
NS = 16
EC = E // NS
EB = 96
NB = 106
ECP = NB * EB
NCH = NP // NS
BLK = 256
NBLK = NP // BLK

_mesh = plsc.VectorSubcoreMesh(
    core_axis_name="c", subcore_axis_name="s", num_cores=NC, num_subcores=NS)

F32 = jnp.float32
I32 = jnp.int32


def _rsqrt16(d):
  i = plsc.bitcast(d, I32)
  y = plsc.bitcast(jnp.int32(0x5F3759DF) - (i >> 1), F32)
  for _ in range(3):
    y = y * (1.5 - 0.5 * d * y * y)
  return y


def _copyeb(src_ref, off, dst_ref, add=None):
  for j in range(EB // 16):
    v = src_ref[pl.ds(off + j * 16, 16)]
    if add is not None:
      v = v + add
    dst_ref[pl.ds(j * 16, 16)] = v


@functools.partial(
    pl.kernel,
    out_type=(
        jax.ShapeDtypeStruct((2 * NP, HHALF), F32),
        jax.ShapeDtypeStruct((NP,), F32),
    ),
    mesh=_mesh,
    compiler_params=pltpu.CompilerParams(needs_layout_passes=False),
    scratch_types=[
        pltpu.VMEM_SHARED((NP,), F32),
        pltpu.VMEM_SHARED((NP,), F32),
        pltpu.VMEM_SHARED((NP,), F32),
        pltpu.VMEM((ECP,), I32),
        pltpu.VMEM((ECP,), I32),
        pltpu.VMEM((EB,), I32),
        pltpu.VMEM((EB,), I32),
        pltpu.VMEM((EB,), F32),
        pltpu.VMEM((EB,), I32),
        pltpu.VMEM((EB,), I32),
        pltpu.VMEM((EB,), F32),
        pltpu.SemaphoreType.DMA,
        pltpu.SemaphoreType.DMA,
        pltpu.VMEM((NCH,), F32),
        pltpu.VMEM((NCH,), F32),
        pltpu.VMEM((NCH,), F32),
        pltpu.VMEM((NCH,), F32),
        pltpu.VMEM((NCH,), F32),
        pltpu.VMEM((HHALF,), F32),
        pltpu.VMEM((HHALF,), F32),
        pltpu.VMEM((32, HHALF), F32),
    ],
)
def _sc_pass_a(src_hbm, dst_hbm, x_hbm, w1_hbm, b1_hbm,
               gcat, dinv_out,
               deg_s, g1_s, s1_s,
               srcb, dstb, idxg, idxs, valb, idxg2, idxs2, valb2,
               sema, semb,
               degc, dinvc, xc, g1c, s1c, w1b, b1b, gout):
  c = lax.axis_index("c")
  s = lax.axis_index("s")
  nslice = pl.ds(s * NCH, NCH)

  padv = jnp.full((16,), N, I32) + (s % 8)
  for k in range(EC, ECP, 16):
    srcb[pl.ds(k, 16)] = padv
    dstb[pl.ds(k, 16)] = padv
  pltpu.sync_copy(src_hbm.at[pl.ds(s * EC, EC)], srcb.at[pl.ds(0, EC)])
  pltpu.sync_copy(dst_hbm.at[pl.ds(s * EC, EC)], dstb.at[pl.ds(0, EC)])

  def zf(k, _):
    degc[pl.ds(k * 16, 16)] = jnp.zeros((16,), F32)
    return 0
  lax.fori_loop(0, NCH // 16, zf, 0)
  pltpu.sync_copy(degc, deg_s.at[nslice])
  pltpu.sync_copy(degc, s1_s.at[nslice])
  for k in range(0, EB, 16):
    valb[pl.ds(k, 16)] = jnp.ones((16,), F32)
    valb2[pl.ds(k, 16)] = jnp.ones((16,), F32)
  plsc.subcore_barrier()

  def deg_it(k, _):
    @pl.when(k > 0)
    def _():
      pltpu.make_async_copy(valb, deg_s.at[idxs], sema).wait()
    _copyeb(dstb, (2 * k) * EB, idxs)
    pltpu.async_copy(valb, deg_s.at[idxs], sema, add=True)
    @pl.when(k > 0)
    def _():
      pltpu.make_async_copy(valb2, deg_s.at[idxs2], semb).wait()
    _copyeb(dstb, (2 * k + 1) * EB, idxs2)
    pltpu.async_copy(valb2, deg_s.at[idxs2], semb, add=True)
    return 0
  lax.fori_loop(0, NB // 2, deg_it, 0)
  pltpu.make_async_copy(valb, deg_s.at[idxs], sema).wait()
  pltpu.make_async_copy(valb2, deg_s.at[idxs2], semb).wait()
  plsc.subcore_barrier()

  pltpu.sync_copy(deg_s.at[nslice], degc)
  pltpu.sync_copy(x_hbm.at[nslice], xc)
  def dv(k, _):
    ks = pl.ds(k * 16, 16)
    y = _rsqrt16(degc[ks] + 1.0)
    dinvc[ks] = y
    g1c[ks] = y * xc[ks]
    return 0
  lax.fori_loop(0, NCH // 16, dv, 0)
  pltpu.sync_copy(g1c, g1_s.at[nslice])
  @pl.when(c == 0)
  def _():
    pltpu.sync_copy(dinvc, dinv_out.at[nslice])
  plsc.subcore_barrier()

  def s1_it(k, _):
    i0 = 2 * k
    @pl.when(k > 0)
    def _():
      pltpu.make_async_copy(valb, s1_s.at[idxs], sema).wait()
    _copyeb(srcb, i0 * EB, idxg)
    pltpu.async_copy(g1_s.at[idxg], valb, sema)
    @pl.when(k > 0)
    def _():
      pltpu.make_async_copy(valb2, s1_s.at[idxs2], semb).wait()
    _copyeb(srcb, (i0 + 1) * EB, idxg2)
    pltpu.async_copy(g1_s.at[idxg2], valb2, semb)
    pltpu.make_async_copy(g1_s.at[idxg], valb, sema).wait()
    _copyeb(dstb, i0 * EB, idxs)
    pltpu.async_copy(valb, s1_s.at[idxs], sema, add=True)
    pltpu.make_async_copy(g1_s.at[idxg2], valb2, semb).wait()
    _copyeb(dstb, (i0 + 1) * EB, idxs2)
    pltpu.async_copy(valb2, s1_s.at[idxs2], semb, add=True)
    return 0
  lax.fori_loop(0, NB // 2, s1_it, 0)
  pltpu.make_async_copy(valb, s1_s.at[idxs], sema).wait()
  pltpu.make_async_copy(valb2, s1_s.at[idxs2], semb).wait()
  plsc.subcore_barrier()

  pltpu.sync_copy(s1_s.at[nslice], s1c)
  pltpu.sync_copy(w1_hbm.at[pl.ds(c * HHALF, HHALF)], w1b)
  pltpu.sync_copy(b1_hbm.at[pl.ds(c * HHALF, HHALF)], b1b)
  def tv(k, _):
    ks = pl.ds(k * 16, 16)
    s1c[ks] = dinvc[ks] * (s1c[ks] + g1c[ks])
    return 0
  lax.fori_loop(0, NCH // 16, tv, 0)
  w1v = [w1b[pl.ds(j * 16, 16)] for j in range(HHALF // 16)]
  b1v = [b1b[pl.ds(j * 16, 16)] for j in range(HHALF // 16)]
  NH8 = 32
  def gchunk(half, _):
    def nv(kb, _):
      tvec = s1c[pl.ds(half * NH8 + kb * 16, 16)]
      dvec = dinvc[pl.ds(half * NH8 + kb * 16, 16)]
      for l in range(16):
        tb = jnp.full((16,), tvec[l], F32)
        db = jnp.full((16,), dvec[l], F32)
        rn = kb * 16 + l
        for j in range(HHALF // 16):
          gout[rn, pl.ds(j * 16, 16)] = (
              jnp.maximum(tb * w1v[j] + b1v[j], 0.0) * db)
      return 0
    lax.fori_loop(0, NH8 // 16, nv, 0)
    pltpu.sync_copy(
        gout, gcat.at[pl.ds(c * NP + s * NCH + half * NH8, NH8), :])
    return 0
  lax.fori_loop(0, NCH // NH8, gchunk, 0)


def _sc_b_scratch():
  return [
      pltpu.VMEM_SHARED((NP, HHALF), F32),
      pltpu.VMEM((ECP,), I32),
      pltpu.VMEM((ECP,), I32),
      pltpu.VMEM((EB,), I32),
      pltpu.VMEM((EB,), I32),
      pltpu.VMEM((EB,), I32),
      pltpu.VMEM((EB,), I32),
      pltpu.VMEM((EB, HHALF), F32),
      pltpu.VMEM((EB, HHALF), F32),
      pltpu.SemaphoreType.DMA,
      pltpu.SemaphoreType.DMA,
      pltpu.SemaphoreType.DMA,
      pltpu.SemaphoreType.DMA,
  ]


def _sc_b_body(issue_gather, wait_gather, c, s,
               src_hbm, dst_hbm, scat,
               acc_s, srcb, dstb, idxga, idxgb, idxs, idxs2, rowsa, rowsb,
               sema, semb, semsa, semsb, coff):
  padv = jnp.full((16,), N, I32) + (s % 8)
  for k in range(EC, ECP, 16):
    srcb[pl.ds(k, 16)] = padv
    dstb[pl.ds(k, 16)] = padv
  pltpu.sync_copy(src_hbm.at[pl.ds(s * EC, EC)], srcb.at[pl.ds(0, EC)])
  pltpu.sync_copy(dst_hbm.at[pl.ds(s * EC, EC)], dstb.at[pl.ds(0, EC)])

  z16 = jnp.zeros((16,), F32)
  for r in range(EB):
    for j in range(HHALF // 16):
      rowsa[r, pl.ds(j * 16, 16)] = z16
  off = 0
  while off < NCH:
    step = min(EB, NCH - off)
    pltpu.sync_copy(rowsa.at[pl.ds(0, step), :],
                    acc_s.at[pl.ds(s * NCH + off, step), :])
    off += step
  plsc.subcore_barrier()

  def ed2(k, _):
    i0 = 2 * k
    @pl.when(k > 0)
    def _():
      pltpu.make_async_copy(rowsa, acc_s.at[idxs], semsa).wait()
    _copyeb(srcb, i0 * EB, idxga, add=coff)
    issue_gather(idxga, rowsa, sema)
    @pl.when(k > 0)
    def _():
      pltpu.make_async_copy(rowsb, acc_s.at[idxs2], semsb).wait()
    _copyeb(srcb, (i0 + 1) * EB, idxgb, add=coff)
    issue_gather(idxgb, rowsb, semb)
    wait_gather(idxga, rowsa, sema)
    _copyeb(dstb, i0 * EB, idxs)
    pltpu.async_copy(rowsa, acc_s.at[idxs], semsa, add=True)
    wait_gather(idxgb, rowsb, semb)
    _copyeb(dstb, (i0 + 1) * EB, idxs2)
    pltpu.async_copy(rowsb, acc_s.at[idxs2], semsb, add=True)
    return 0
  lax.fori_loop(0, NB // 2, ed2, 0)
  pltpu.make_async_copy(rowsa, acc_s.at[idxs], semsa).wait()
  pltpu.make_async_copy(rowsb, acc_s.at[idxs2], semsb).wait()
  plsc.subcore_barrier()

  pltpu.sync_copy(acc_s.at[pl.ds(s * NCH, NCH), :],
                  scat.at[pl.ds(c * NP + s * NCH, NCH), :])


@functools.partial(
    pl.kernel,
    out_type=jax.ShapeDtypeStruct((2 * NP, HHALF), F32),
    mesh=_mesh,
    compiler_params=pltpu.CompilerParams(needs_layout_passes=False),
    scratch_types=_sc_b_scratch(),
)
def _sc_pass_b_cat(src_hbm, dst_hbm, gcat, scat, *scr):
  c = lax.axis_index("c")
  s = lax.axis_index("s")
  coff = jnp.full((16,), 0, I32) + c * NP

  def issue(idxref, rowsref, sem):
    pltpu.async_copy(gcat.at[idxref], rowsref, sem)

  def wait(idxref, rowsref, sem):
    pltpu.make_async_copy(gcat.at[idxref], rowsref, sem).wait()

  _sc_b_body(issue, wait, c, s, src_hbm, dst_hbm, scat, *scr, coff=coff)


@functools.partial(
    pl.kernel,
    out_type=jax.ShapeDtypeStruct((2 * NP, HHALF), F32),
    mesh=_mesh,
    compiler_params=pltpu.CompilerParams(needs_layout_passes=False),
    scratch_types=_sc_b_scratch(),
)
def _sc_pass_b_ab(src_hbm, dst_hbm, ga, gb, scat, *scr):
  c = lax.axis_index("c")
  s = lax.axis_index("s")
  coff = jnp.full((16,), 0, I32)

  def issue(idxref, rowsref, sem):
    @pl.when(c == 0)
    def _():
      pltpu.async_copy(ga.at[idxref], rowsref, sem)
    @pl.when(c == 1)
    def _():
      pltpu.async_copy(gb.at[idxref], rowsref, sem)

  def wait(idxref, rowsref, sem):
    pltpu.make_async_copy(ga.at[idxref], rowsref, sem).wait()

  _sc_b_body(issue, wait, c, s, src_hbm, dst_hbm, scat, *scr, coff=coff)


def _tc_layer_body(dinv_ref, sa_ref, sb_ref, ga_ref, gb_ref, w_ref, b_ref,
                   oa_ref, ob_ref):
  dv = dinv_ref[...]
  t = jnp.concatenate(
      [sa_ref[...] + ga_ref[...], sb_ref[...] + gb_ref[...]], axis=1) * dv
  h = jnp.maximum(
      jnp.dot(t, w_ref[...], preferred_element_type=F32) + b_ref[...], 0.0)
  gn = h * dv
  oa_ref[...] = gn[:, :HHALF]
  ob_ref[...] = gn[:, HHALF:]


_tc_layer = pl.pallas_call(
    _tc_layer_body,
    grid=(NBLK,),
    in_specs=[
        pl.BlockSpec((BLK, 1), lambda i: (i, 0)),
        pl.BlockSpec((BLK, HHALF), lambda i: (i, 0)),
        pl.BlockSpec((BLK, HHALF), lambda i: (NBLK + i, 0)),
        pl.BlockSpec((BLK, HHALF), lambda i: (i, 0)),
        pl.BlockSpec((BLK, HHALF), lambda i: (NBLK + i, 0)),
        pl.BlockSpec((HID, HID), lambda i: (0, 0)),
        pl.BlockSpec((1, HID), lambda i: (0, 0)),
    ],
    out_specs=[
        pl.BlockSpec((BLK, HHALF), lambda i: (i, 0)),
        pl.BlockSpec((BLK, HHALF), lambda i: (i, 0)),
    ],
    out_shape=[
        jax.ShapeDtypeStruct((NP, HHALF), F32),
        jax.ShapeDtypeStruct((NP, HHALF), F32),
    ],
)


def _tc_final_body(dinv_ref, sa_ref, sb_ref, ga_ref, gb_ref, w3_ref, b3_ref,
                   batch_ref, nd_ref, m1w_ref, m1b_ref, m2w_ref, m2b_ref,
                   m3w_ref, m3b_ref, fw_ref, fb_ref,
                   out_ref, pool_acc, cnt_acc):
  i = pl.program_id(0)
  dv = dinv_ref[...]
  t = jnp.concatenate(
      [sa_ref[...] + ga_ref[...], sb_ref[...] + gb_ref[...]], axis=1) * dv
  h = jnp.maximum(
      jnp.dot(t, w3_ref[...], preferred_element_type=F32) + b3_ref[...], 0.0)
  onehot = (lax.broadcasted_iota(I32, (G, BLK), 0)
            == batch_ref[...]).astype(F32)
  psum = jnp.dot(onehot, h, preferred_element_type=F32)
  pcnt = jnp.sum(onehot, axis=1, keepdims=True)

  @pl.when(i == 0)
  def _():
    pool_acc[...] = jnp.zeros_like(pool_acc)
    cnt_acc[...] = jnp.zeros_like(cnt_acc)

  pool_acc[...] += psum
  cnt_acc[...] += pcnt

  @pl.when(i == NBLK - 1)
  def _():
    emb = pool_acc[...] / jnp.maximum(cnt_acc[...], 1.0)
    z = jnp.maximum(
        jnp.dot(nd_ref[...], m1w_ref[...], preferred_element_type=F32)
        + m1b_ref[...], 0.0)
    z = jnp.maximum(
        jnp.dot(z, m2w_ref[...], preferred_element_type=F32)
        + m2b_ref[...], 0.0)
    z = jnp.dot(z, m3w_ref[...], preferred_element_type=F32) + m3b_ref[...]
    res = (jnp.dot(emb, fw_ref[:HID, :], preferred_element_type=F32)
           + jnp.dot(z, fw_ref[HID:, :], preferred_element_type=F32)
           + fb_ref[...])
    out_ref[...] = res


_tc_final = pl.pallas_call(
    _tc_final_body,
    grid=(NBLK,),
    in_specs=[
        pl.BlockSpec((BLK, 1), lambda i: (i, 0)),
        pl.BlockSpec((BLK, HHALF), lambda i: (i, 0)),
        pl.BlockSpec((BLK, HHALF), lambda i: (NBLK + i, 0)),
        pl.BlockSpec((BLK, HHALF), lambda i: (i, 0)),
        pl.BlockSpec((BLK, HHALF), lambda i: (i, 0)),
        pl.BlockSpec((HID, HID), lambda i: (0, 0)),
        pl.BlockSpec((1, HID), lambda i: (0, 0)),
        pl.BlockSpec((1, BLK), lambda i: (0, i)),
        pl.BlockSpec((G, MLP_IN), lambda i: (0, 0)),
        pl.BlockSpec((MLP_IN, HID), lambda i: (0, 0)),
        pl.BlockSpec((1, HID), lambda i: (0, 0)),
        pl.BlockSpec((HID, HID), lambda i: (0, 0)),
        pl.BlockSpec((1, HID), lambda i: (0, 0)),
        pl.BlockSpec((HID, HID), lambda i: (0, 0)),
        pl.BlockSpec((1, HID), lambda i: (0, 0)),
        pl.BlockSpec((HID + HID, OUT), lambda i: (0, 0)),
        pl.BlockSpec((1, OUT), lambda i: (0, 0)),
    ],
    out_specs=pl.BlockSpec((G, OUT), lambda i: (0, 0)),
    out_shape=jax.ShapeDtypeStruct((G, OUT), F32),
    scratch_shapes=[
        pltpu.VMEM((G, HID), F32),
        pltpu.VMEM((G, 1), F32),
    ],
)


def kernel(x, edge_index, batch, numerical_data, W1, b1, W2, b2, W3, b3,
           M1w, M1b, M2w, M2b, M3w, M3b, Fw, Fb):
  src = edge_index[0]
  dst = edge_index[1]
  xpad = jnp.pad(x[:, 0], (0, NP - N))
  batch2d = jnp.pad(batch, (0, NP - N), constant_values=G).reshape(1, NP)

  gcat, dinv = _sc_pass_a(src, dst, xpad, W1.reshape(HID), b1)
  dinv2d = dinv.reshape(NP, 1)

  scat = _sc_pass_b_cat(src, dst, gcat)
  g2a, g2b = _tc_layer(dinv2d, scat, scat, gcat, gcat, W2,
                       b2.reshape(1, HID))

  scat3 = _sc_pass_b_ab(src, dst, g2a, g2b)
  out = _tc_final(dinv2d, scat3, scat3, g2a, g2b, W3, b3.reshape(1, HID),
                  batch2d, numerical_data,
                  M1w, M1b.reshape(1, HID), M2w, M2b.reshape(1, HID),
                  M3w, M3b.reshape(1, HID), Fw, Fb.reshape(1, OUT))
  return out


# speedup vs baseline: 1.2360x; 1.2360x over previous
import functools

import jax
import jax.numpy as jnp
from jax import lax
from jax.experimental import pallas as pl
from jax.experimental.pallas import tpu as pltpu
from jax.experimental.pallas import tpu_sc as plsc

N = 10000
NP = 10240
E = 160000
G = 512
HID = 256
HHALF = 128
MLP_IN = 128
OUT = 128

NC = 2
NS = 16
EC = E // NS
EB = 96
NB = 106
ECP = NB * EB
NCH = NP // NS
BLK = 256
NBLK = NP // BLK
BLKF = 512
NBLKF = NP // BLKF

_mesh = plsc.VectorSubcoreMesh(
    core_axis_name="c", subcore_axis_name="s", num_cores=NC, num_subcores=NS)

F32 = jnp.float32
I32 = jnp.int32


def _rsqrt16(d):
  i = plsc.bitcast(d, I32)
  y = plsc.bitcast(jnp.int32(0x5F3759DF) - (i >> 1), F32)
  for _ in range(3):
    y = y * (1.5 - 0.5 * d * y * y)
  return y


def _copyeb(src_ref, off, dst_ref, add=None):
  for j in range(EB // 16):
    v = src_ref[pl.ds(off + j * 16, 16)]
    if add is not None:
      v = v + add
    dst_ref[pl.ds(j * 16, 16)] = v


@functools.partial(
    pl.kernel,
    out_type=(
        jax.ShapeDtypeStruct((2 * NP, HHALF), F32),
        jax.ShapeDtypeStruct((NP,), F32),
    ),
    mesh=_mesh,
    compiler_params=pltpu.CompilerParams(needs_layout_passes=False),
    scratch_types=[
        pltpu.VMEM_SHARED((NP,), F32),
        pltpu.VMEM_SHARED((NP,), F32),
        pltpu.VMEM_SHARED((NP,), F32),
        pltpu.VMEM((ECP,), I32),
        pltpu.VMEM((ECP,), I32),
        pltpu.VMEM((EB,), I32),
        pltpu.VMEM((EB,), I32),
        pltpu.VMEM((EB,), F32),
        pltpu.VMEM((EB,), I32),
        pltpu.VMEM((EB,), I32),
        pltpu.VMEM((EB,), F32),
        pltpu.SemaphoreType.DMA,
        pltpu.SemaphoreType.DMA,
        pltpu.VMEM((NCH,), F32),
        pltpu.VMEM((NCH,), F32),
        pltpu.VMEM((NCH,), F32),
        pltpu.VMEM((NCH,), F32),
        pltpu.VMEM((NCH,), F32),
        pltpu.VMEM((HHALF,), F32),
        pltpu.VMEM((HHALF,), F32),
        pltpu.VMEM((32, HHALF), F32),
    ],
)
def _sc_pass_a(src_hbm, dst_hbm, x_hbm, w1_hbm, b1_hbm,
               gcat, dinv_out,
               deg_s, g1_s, s1_s,
               srcb, dstb, idxg, idxs, valb, idxg2, idxs2, valb2,
               sema, semb,
               degc, dinvc, xc, g1c, s1c, w1b, b1b, gout):
  c = lax.axis_index("c")
  s = lax.axis_index("s")
  nslice = pl.ds(s * NCH, NCH)

  padv = jnp.full((16,), N, I32) + (s % 8)
  for k in range(EC, ECP, 16):
    srcb[pl.ds(k, 16)] = padv
    dstb[pl.ds(k, 16)] = padv
  pltpu.sync_copy(src_hbm.at[pl.ds(s * EC, EC)], srcb.at[pl.ds(0, EC)])
  pltpu.sync_copy(dst_hbm.at[pl.ds(s * EC, EC)], dstb.at[pl.ds(0, EC)])

  def zf(k, _):
    degc[pl.ds(k * 16, 16)] = jnp.zeros((16,), F32)
    return 0
  lax.fori_loop(0, NCH // 16, zf, 0)
  pltpu.sync_copy(degc, deg_s.at[nslice])
  pltpu.sync_copy(degc, s1_s.at[nslice])
  for k in range(0, EB, 16):
    valb[pl.ds(k, 16)] = jnp.ones((16,), F32)
    valb2[pl.ds(k, 16)] = jnp.ones((16,), F32)
  plsc.subcore_barrier()

  def deg_it(k, _):
    @pl.when(k > 0)
    def _():
      pltpu.make_async_copy(valb, deg_s.at[idxs], sema).wait()
    _copyeb(dstb, (2 * k) * EB, idxs)
    pltpu.async_copy(valb, deg_s.at[idxs], sema, add=True)
    @pl.when(k > 0)
    def _():
      pltpu.make_async_copy(valb2, deg_s.at[idxs2], semb).wait()
    _copyeb(dstb, (2 * k + 1) * EB, idxs2)
    pltpu.async_copy(valb2, deg_s.at[idxs2], semb, add=True)
    return 0
  lax.fori_loop(0, NB // 2, deg_it, 0)
  pltpu.make_async_copy(valb, deg_s.at[idxs], sema).wait()
  pltpu.make_async_copy(valb2, deg_s.at[idxs2], semb).wait()
  plsc.subcore_barrier()

  pltpu.sync_copy(deg_s.at[nslice], degc)
  pltpu.sync_copy(x_hbm.at[nslice], xc)
  def dv(k, _):
    ks = pl.ds(k * 16, 16)
    y = _rsqrt16(degc[ks] + 1.0)
    dinvc[ks] = y
    g1c[ks] = y * xc[ks]
    return 0
  lax.fori_loop(0, NCH // 16, dv, 0)
  pltpu.sync_copy(g1c, g1_s.at[nslice])
  @pl.when(c == 0)
  def _():
    pltpu.sync_copy(dinvc, dinv_out.at[nslice])
  plsc.subcore_barrier()

  def s1_it(k, _):
    i0 = 2 * k
    @pl.when(k > 0)
    def _():
      pltpu.make_async_copy(valb, s1_s.at[idxs], sema).wait()
    _copyeb(srcb, i0 * EB, idxg)
    pltpu.async_copy(g1_s.at[idxg], valb, sema)
    @pl.when(k > 0)
    def _():
      pltpu.make_async_copy(valb2, s1_s.at[idxs2], semb).wait()
    _copyeb(srcb, (i0 + 1) * EB, idxg2)
    pltpu.async_copy(g1_s.at[idxg2], valb2, semb)
    pltpu.make_async_copy(g1_s.at[idxg], valb, sema).wait()
    _copyeb(dstb, i0 * EB, idxs)
    pltpu.async_copy(valb, s1_s.at[idxs], sema, add=True)
    pltpu.make_async_copy(g1_s.at[idxg2], valb2, semb).wait()
    _copyeb(dstb, (i0 + 1) * EB, idxs2)
    pltpu.async_copy(valb2, s1_s.at[idxs2], semb, add=True)
    return 0
  lax.fori_loop(0, NB // 2, s1_it, 0)
  pltpu.make_async_copy(valb, s1_s.at[idxs], sema).wait()
  pltpu.make_async_copy(valb2, s1_s.at[idxs2], semb).wait()
  plsc.subcore_barrier()

  pltpu.sync_copy(s1_s.at[nslice], s1c)
  pltpu.sync_copy(w1_hbm.at[pl.ds(c * HHALF, HHALF)], w1b)
  pltpu.sync_copy(b1_hbm.at[pl.ds(c * HHALF, HHALF)], b1b)
  def tv(k, _):
    ks = pl.ds(k * 16, 16)
    s1c[ks] = dinvc[ks] * (s1c[ks] + g1c[ks])
    return 0
  lax.fori_loop(0, NCH // 16, tv, 0)
  w1v = [w1b[pl.ds(j * 16, 16)] for j in range(HHALF // 16)]
  b1v = [b1b[pl.ds(j * 16, 16)] for j in range(HHALF // 16)]
  NH8 = 32
  def gchunk(half, _):
    def nv(kb, _):
      tvec = s1c[pl.ds(half * NH8 + kb * 16, 16)]
      dvec = dinvc[pl.ds(half * NH8 + kb * 16, 16)]
      for l in range(16):
        tb = jnp.full((16,), tvec[l], F32)
        db = jnp.full((16,), dvec[l], F32)
        rn = kb * 16 + l
        for j in range(HHALF // 16):
          gout[rn, pl.ds(j * 16, 16)] = (
              jnp.maximum(tb * w1v[j] + b1v[j], 0.0) * db)
      return 0
    lax.fori_loop(0, NH8 // 16, nv, 0)
    pltpu.sync_copy(
        gout, gcat.at[pl.ds(c * NP + s * NCH + half * NH8, NH8), :])
    return 0
  lax.fori_loop(0, NCH // NH8, gchunk, 0)


def _sc_b_scratch():
  return [
      pltpu.VMEM_SHARED((NP, HHALF), F32),
      pltpu.VMEM((ECP,), I32),
      pltpu.VMEM((ECP,), I32),
      pltpu.VMEM((EB,), I32),
      pltpu.VMEM((EB,), I32),
      pltpu.VMEM((EB,), I32),
      pltpu.VMEM((EB,), I32),
      pltpu.VMEM((EB, HHALF), F32),
      pltpu.VMEM((EB, HHALF), F32),
      pltpu.SemaphoreType.DMA,
      pltpu.SemaphoreType.DMA,
      pltpu.SemaphoreType.DMA,
      pltpu.SemaphoreType.DMA,
  ]


def _sc_b_body(issue_gather, wait_gather, c, s,
               src_hbm, dst_hbm, scat,
               acc_s, srcb, dstb, idxga, idxgb, idxs, idxs2, rowsa, rowsb,
               sema, semb, semsa, semsb, coff):
  padv = jnp.full((16,), N, I32) + (s % 8)
  for k in range(EC, ECP, 16):
    srcb[pl.ds(k, 16)] = padv
    dstb[pl.ds(k, 16)] = padv
  pltpu.sync_copy(src_hbm.at[pl.ds(s * EC, EC)], srcb.at[pl.ds(0, EC)])
  pltpu.sync_copy(dst_hbm.at[pl.ds(s * EC, EC)], dstb.at[pl.ds(0, EC)])

  z16 = jnp.zeros((16,), F32)
  for r in range(EB):
    for j in range(HHALF // 16):
      rowsa[r, pl.ds(j * 16, 16)] = z16
  off = 0
  while off < NCH:
    step = min(EB, NCH - off)
    pltpu.sync_copy(rowsa.at[pl.ds(0, step), :],
                    acc_s.at[pl.ds(s * NCH + off, step), :])
    off += step
  plsc.subcore_barrier()

  def ed2(k, _):
    i0 = 2 * k
    _copyeb(srcb, i0 * EB, idxga, add=coff)
    issue_gather(idxga, rowsa, sema)
    @pl.when(k > 0)
    def _():
      wait_gather(idxgb, rowsb, semb)
      _copyeb(dstb, (i0 - 1) * EB, idxs)
      pltpu.sync_copy(rowsb, acc_s.at[idxs], add=True)
    _copyeb(srcb, (i0 + 1) * EB, idxgb, add=coff)
    issue_gather(idxgb, rowsb, semb)
    wait_gather(idxga, rowsa, sema)
    _copyeb(dstb, i0 * EB, idxs)
    pltpu.sync_copy(rowsa, acc_s.at[idxs], add=True)
    return 0
  lax.fori_loop(0, NB // 2, ed2, 0)
  wait_gather(idxgb, rowsb, semb)
  _copyeb(dstb, (NB - 1) * EB, idxs)
  pltpu.sync_copy(rowsb, acc_s.at[idxs], add=True)
  plsc.subcore_barrier()

  pltpu.sync_copy(acc_s.at[pl.ds(s * NCH, NCH), :],
                  scat.at[pl.ds(c * NP + s * NCH, NCH), :])


@functools.partial(
    pl.kernel,
    out_type=jax.ShapeDtypeStruct((2 * NP, HHALF), F32),
    mesh=_mesh,
    compiler_params=pltpu.CompilerParams(needs_layout_passes=False),
    scratch_types=_sc_b_scratch(),
)
def _sc_pass_b_cat(src_hbm, dst_hbm, gcat, scat, *scr):
  c = lax.axis_index("c")
  s = lax.axis_index("s")
  coff = jnp.full((16,), 0, I32) + c * NP

  def issue(idxref, rowsref, sem):
    pltpu.async_copy(gcat.at[idxref], rowsref, sem)

  def wait(idxref, rowsref, sem):
    pltpu.make_async_copy(gcat.at[idxref], rowsref, sem).wait()

  _sc_b_body(issue, wait, c, s, src_hbm, dst_hbm, scat, *scr, coff=coff)


@functools.partial(
    pl.kernel,
    out_type=jax.ShapeDtypeStruct((2 * NP, HHALF), F32),
    mesh=_mesh,
    compiler_params=pltpu.CompilerParams(needs_layout_passes=False),
    scratch_types=_sc_b_scratch(),
)
def _sc_pass_b_ab(src_hbm, dst_hbm, ga, gb, scat, *scr):
  c = lax.axis_index("c")
  s = lax.axis_index("s")
  coff = jnp.full((16,), 0, I32)

  def issue(idxref, rowsref, sem):
    @pl.when(c == 0)
    def _():
      pltpu.async_copy(ga.at[idxref], rowsref, sem)
    @pl.when(c == 1)
    def _():
      pltpu.async_copy(gb.at[idxref], rowsref, sem)

  def wait(idxref, rowsref, sem):
    pltpu.make_async_copy(ga.at[idxref], rowsref, sem).wait()

  _sc_b_body(issue, wait, c, s, src_hbm, dst_hbm, scat, *scr, coff=coff)


def _tc_layer_body(dinv_ref, sa_ref, sb_ref, ga_ref, gb_ref, w_ref, b_ref,
                   oa_ref, ob_ref):
  dv = dinv_ref[...]
  t = jnp.concatenate(
      [sa_ref[...] + ga_ref[...], sb_ref[...] + gb_ref[...]], axis=1) * dv
  h = jnp.maximum(
      jnp.dot(t, w_ref[...], preferred_element_type=F32) + b_ref[...], 0.0)
  gn = h * dv
  oa_ref[...] = gn[:, :HHALF]
  ob_ref[...] = gn[:, HHALF:]


_tc_layer = pl.pallas_call(
    _tc_layer_body,
    grid=(NBLK,),
    in_specs=[
        pl.BlockSpec((BLK, 1), lambda i: (i, 0)),
        pl.BlockSpec((BLK, HHALF), lambda i: (i, 0)),
        pl.BlockSpec((BLK, HHALF), lambda i: (NBLK + i, 0)),
        pl.BlockSpec((BLK, HHALF), lambda i: (i, 0)),
        pl.BlockSpec((BLK, HHALF), lambda i: (NBLK + i, 0)),
        pl.BlockSpec((HID, HID), lambda i: (0, 0)),
        pl.BlockSpec((1, HID), lambda i: (0, 0)),
    ],
    out_specs=[
        pl.BlockSpec((BLK, HHALF), lambda i: (i, 0)),
        pl.BlockSpec((BLK, HHALF), lambda i: (i, 0)),
    ],
    out_shape=[
        jax.ShapeDtypeStruct((NP, HHALF), F32),
        jax.ShapeDtypeStruct((NP, HHALF), F32),
    ],
)


def _tc_final_body(dinv_ref, sa_ref, sb_ref, ga_ref, gb_ref, w3_ref, b3_ref,
                   batch_ref, nd_ref, m1w_ref, m1b_ref, m2w_ref, m2b_ref,
                   m3w_ref, m3b_ref, fw_ref, fb_ref,
                   out_ref, pool_acc, cnt_acc):
  i = pl.program_id(0)
  dv = dinv_ref[...]
  t = jnp.concatenate(
      [sa_ref[...] + ga_ref[...], sb_ref[...] + gb_ref[...]], axis=1) * dv
  h = jnp.maximum(
      jnp.dot(t, w3_ref[...], preferred_element_type=F32) + b3_ref[...], 0.0)
  onehot = (lax.broadcasted_iota(I32, (G, BLKF), 0)
            == batch_ref[...]).astype(F32)
  psum = jnp.dot(onehot, h, preferred_element_type=F32)
  pcnt = jnp.sum(onehot, axis=1, keepdims=True)

  @pl.when(i == 0)
  def _():
    pool_acc[...] = jnp.zeros_like(pool_acc)
    cnt_acc[...] = jnp.zeros_like(cnt_acc)

  pool_acc[...] += psum
  cnt_acc[...] += pcnt

  @pl.when(i == NBLKF - 1)
  def _():
    emb = pool_acc[...] / jnp.maximum(cnt_acc[...], 1.0)
    z = jnp.maximum(
        jnp.dot(nd_ref[...], m1w_ref[...], preferred_element_type=F32)
        + m1b_ref[...], 0.0)
    z = jnp.maximum(
        jnp.dot(z, m2w_ref[...], preferred_element_type=F32)
        + m2b_ref[...], 0.0)
    z = jnp.dot(z, m3w_ref[...], preferred_element_type=F32) + m3b_ref[...]
    res = (jnp.dot(emb, fw_ref[:HID, :], preferred_element_type=F32)
           + jnp.dot(z, fw_ref[HID:, :], preferred_element_type=F32)
           + fb_ref[...])
    out_ref[...] = res


_tc_final = pl.pallas_call(
    _tc_final_body,
    grid=(NBLKF,),
    in_specs=[
        pl.BlockSpec((BLKF, 1), lambda i: (i, 0)),
        pl.BlockSpec((BLKF, HHALF), lambda i: (i, 0)),
        pl.BlockSpec((BLKF, HHALF), lambda i: (NBLKF + i, 0)),
        pl.BlockSpec((BLKF, HHALF), lambda i: (i, 0)),
        pl.BlockSpec((BLKF, HHALF), lambda i: (i, 0)),
        pl.BlockSpec((HID, HID), lambda i: (0, 0)),
        pl.BlockSpec((1, HID), lambda i: (0, 0)),
        pl.BlockSpec((1, BLKF), lambda i: (0, i)),
        pl.BlockSpec((G, MLP_IN), lambda i: (0, 0)),
        pl.BlockSpec((MLP_IN, HID), lambda i: (0, 0)),
        pl.BlockSpec((1, HID), lambda i: (0, 0)),
        pl.BlockSpec((HID, HID), lambda i: (0, 0)),
        pl.BlockSpec((1, HID), lambda i: (0, 0)),
        pl.BlockSpec((HID, HID), lambda i: (0, 0)),
        pl.BlockSpec((1, HID), lambda i: (0, 0)),
        pl.BlockSpec((HID + HID, OUT), lambda i: (0, 0)),
        pl.BlockSpec((1, OUT), lambda i: (0, 0)),
    ],
    out_specs=pl.BlockSpec((G, OUT), lambda i: (0, 0)),
    out_shape=jax.ShapeDtypeStruct((G, OUT), F32),
    scratch_shapes=[
        pltpu.VMEM((G, HID), F32),
        pltpu.VMEM((G, 1), F32),
    ],
)


def kernel(x, edge_index, batch, numerical_data, W1, b1, W2, b2, W3, b3,
           M1w, M1b, M2w, M2b, M3w, M3b, Fw, Fb):
  src = edge_index[0]
  dst = edge_index[1]
  xpad = jnp.pad(x[:, 0], (0, NP - N))
  batch2d = jnp.pad(batch, (0, NP - N), constant_values=G).reshape(1, NP)

  gcat, dinv = _sc_pass_a(src, dst, xpad, W1.reshape(HID), b1)
  dinv2d = dinv.reshape(NP, 1)

  scat = _sc_pass_b_cat(src, dst, gcat)
  g2a, g2b = _tc_layer(dinv2d, scat, scat, gcat, gcat, W2,
                       b2.reshape(1, HID))

  scat3 = _sc_pass_b_ab(src, dst, g2a, g2b)
  out = _tc_final(dinv2d, scat3, scat3, g2a, g2b, W3, b3.reshape(1, HID),
                  batch2d, numerical_data,
                  M1w, M1b.reshape(1, HID), M2w, M2b.reshape(1, HID),
                  M3w, M3b.reshape(1, HID), Fw, Fb.reshape(1, OUT))
  return out


# speedup vs baseline: 1.2719x; 1.0290x over previous
import functools

import jax
import jax.numpy as jnp
from jax import lax
from jax.experimental import pallas as pl
from jax.experimental.pallas import tpu as pltpu
from jax.experimental.pallas import tpu_sc as plsc

N = 10000
NP = 10240
E = 160000
G = 512
HID = 256
HHALF = 128
MLP_IN = 128
OUT = 128

NC = 2
NS = 16
EC = E // NS
EB = 96
NB = 106
EBS = 128
NBS = 80
ECP = NBS * EBS
NCH = NP // NS
BLK = 256
NBLK = NP // BLK
BLKF = 512
NBLKF = NP // BLKF

_mesh = plsc.VectorSubcoreMesh(
    core_axis_name="c", subcore_axis_name="s", num_cores=NC, num_subcores=NS)

F32 = jnp.float32
I32 = jnp.int32


def _rsqrt16(d):
  i = plsc.bitcast(d, I32)
  y = plsc.bitcast(jnp.int32(0x5F3759DF) - (i >> 1), F32)
  for _ in range(3):
    y = y * (1.5 - 0.5 * d * y * y)
  return y


def _copyn(n, src_ref, off, dst_ref, add=None):
  for j in range(n // 16):
    v = src_ref[pl.ds(off + j * 16, 16)]
    if add is not None:
      v = v + add
    dst_ref[pl.ds(j * 16, 16)] = v


def _copyeb(src_ref, off, dst_ref, add=None):
  _copyn(EB, src_ref, off, dst_ref, add=add)


@functools.partial(
    pl.kernel,
    out_type=(
        jax.ShapeDtypeStruct((2 * NP, HHALF), F32),
        jax.ShapeDtypeStruct((NP,), F32),
    ),
    mesh=_mesh,
    compiler_params=pltpu.CompilerParams(needs_layout_passes=False),
    scratch_types=[
        pltpu.VMEM_SHARED((NP,), F32),
        pltpu.VMEM_SHARED((NP,), F32),
        pltpu.VMEM_SHARED((NP,), F32),
        pltpu.VMEM((ECP,), I32),
        pltpu.VMEM((ECP,), I32),
        pltpu.VMEM((EBS,), I32),
        pltpu.VMEM((EBS,), I32),
        pltpu.VMEM((EBS,), F32),
        pltpu.VMEM((EBS,), I32),
        pltpu.VMEM((EBS,), I32),
        pltpu.VMEM((EBS,), F32),
        pltpu.SemaphoreType.DMA,
        pltpu.SemaphoreType.DMA,
        pltpu.VMEM((NCH,), F32),
        pltpu.VMEM((NCH,), F32),
        pltpu.VMEM((NCH,), F32),
        pltpu.VMEM((NCH,), F32),
        pltpu.VMEM((NCH,), F32),
        pltpu.VMEM((HHALF,), F32),
        pltpu.VMEM((HHALF,), F32),
        pltpu.VMEM((32, HHALF), F32),
    ],
)
def _sc_pass_a(src_hbm, dst_hbm, x_hbm, w1_hbm, b1_hbm,
               gcat, dinv_out,
               deg_s, g1_s, s1_s,
               srcb, dstb, idxg, idxs, valb, idxg2, idxs2, valb2,
               sema, semb,
               degc, dinvc, xc, g1c, s1c, w1b, b1b, gout):
  c = lax.axis_index("c")
  s = lax.axis_index("s")
  nslice = pl.ds(s * NCH, NCH)

  padv = jnp.full((16,), N, I32) + (s % 8)
  for k in range(EC, ECP, 16):
    srcb[pl.ds(k, 16)] = padv
    dstb[pl.ds(k, 16)] = padv
  pltpu.sync_copy(src_hbm.at[pl.ds(s * EC, EC)], srcb.at[pl.ds(0, EC)])
  pltpu.sync_copy(dst_hbm.at[pl.ds(s * EC, EC)], dstb.at[pl.ds(0, EC)])

  def zf(k, _):
    degc[pl.ds(k * 16, 16)] = jnp.zeros((16,), F32)
    return 0
  lax.fori_loop(0, NCH // 16, zf, 0)
  pltpu.sync_copy(degc, deg_s.at[nslice])
  pltpu.sync_copy(degc, s1_s.at[nslice])
  for k in range(0, EBS, 16):
    valb[pl.ds(k, 16)] = jnp.ones((16,), F32)
    valb2[pl.ds(k, 16)] = jnp.ones((16,), F32)
  plsc.subcore_barrier()

  def deg_it(k, _):
    @pl.when(k > 0)
    def _():
      pltpu.make_async_copy(valb, deg_s.at[idxs], sema).wait()
    _copyn(EBS, dstb, (2 * k) * EBS, idxs)
    pltpu.async_copy(valb, deg_s.at[idxs], sema, add=True)
    @pl.when(k > 0)
    def _():
      pltpu.make_async_copy(valb2, deg_s.at[idxs2], semb).wait()
    _copyn(EBS, dstb, (2 * k + 1) * EBS, idxs2)
    pltpu.async_copy(valb2, deg_s.at[idxs2], semb, add=True)
    return 0
  lax.fori_loop(0, NBS // 2, deg_it, 0)
  pltpu.make_async_copy(valb, deg_s.at[idxs], sema).wait()
  pltpu.make_async_copy(valb2, deg_s.at[idxs2], semb).wait()
  plsc.subcore_barrier()

  pltpu.sync_copy(deg_s.at[nslice], degc)
  pltpu.sync_copy(x_hbm.at[nslice], xc)
  def dv(k, _):
    ks = pl.ds(k * 16, 16)
    y = _rsqrt16(degc[ks] + 1.0)
    dinvc[ks] = y
    g1c[ks] = y * xc[ks]
    return 0
  lax.fori_loop(0, NCH // 16, dv, 0)
  pltpu.sync_copy(g1c, g1_s.at[nslice])
  @pl.when(c == 0)
  def _():
    pltpu.sync_copy(dinvc, dinv_out.at[nslice])
  plsc.subcore_barrier()

  def s1_it(k, _):
    i0 = 2 * k
    @pl.when(k > 0)
    def _():
      pltpu.make_async_copy(valb, s1_s.at[idxs], sema).wait()
    _copyn(EBS, srcb, i0 * EBS, idxg)
    pltpu.async_copy(g1_s.at[idxg], valb, sema)
    @pl.when(k > 0)
    def _():
      pltpu.make_async_copy(valb2, s1_s.at[idxs2], semb).wait()
    _copyn(EBS, srcb, (i0 + 1) * EBS, idxg2)
    pltpu.async_copy(g1_s.at[idxg2], valb2, semb)
    pltpu.make_async_copy(g1_s.at[idxg], valb, sema).wait()
    _copyn(EBS, dstb, i0 * EBS, idxs)
    pltpu.async_copy(valb, s1_s.at[idxs], sema, add=True)
    pltpu.make_async_copy(g1_s.at[idxg2], valb2, semb).wait()
    _copyn(EBS, dstb, (i0 + 1) * EBS, idxs2)
    pltpu.async_copy(valb2, s1_s.at[idxs2], semb, add=True)
    return 0
  lax.fori_loop(0, NBS // 2, s1_it, 0)
  pltpu.make_async_copy(valb, s1_s.at[idxs], sema).wait()
  pltpu.make_async_copy(valb2, s1_s.at[idxs2], semb).wait()
  plsc.subcore_barrier()

  pltpu.sync_copy(s1_s.at[nslice], s1c)
  pltpu.sync_copy(w1_hbm.at[pl.ds(c * HHALF, HHALF)], w1b)
  pltpu.sync_copy(b1_hbm.at[pl.ds(c * HHALF, HHALF)], b1b)
  def tv(k, _):
    ks = pl.ds(k * 16, 16)
    s1c[ks] = dinvc[ks] * (s1c[ks] + g1c[ks])
    return 0
  lax.fori_loop(0, NCH // 16, tv, 0)
  w1v = [w1b[pl.ds(j * 16, 16)] for j in range(HHALF // 16)]
  b1v = [b1b[pl.ds(j * 16, 16)] for j in range(HHALF // 16)]
  NH8 = 32
  def gchunk(half, _):
    def nv(kb, _):
      tvec = s1c[pl.ds(half * NH8 + kb * 16, 16)]
      dvec = dinvc[pl.ds(half * NH8 + kb * 16, 16)]
      for l in range(16):
        tb = jnp.full((16,), tvec[l], F32)
        db = jnp.full((16,), dvec[l], F32)
        rn = kb * 16 + l
        for j in range(HHALF // 16):
          gout[rn, pl.ds(j * 16, 16)] = (
              jnp.maximum(tb * w1v[j] + b1v[j], 0.0) * db)
      return 0
    lax.fori_loop(0, NH8 // 16, nv, 0)
    pltpu.sync_copy(
        gout, gcat.at[pl.ds(c * NP + s * NCH + half * NH8, NH8), :])
    return 0
  lax.fori_loop(0, NCH // NH8, gchunk, 0)


def _sc_b_scratch():
  return [
      pltpu.VMEM_SHARED((NP, HHALF), F32),
      pltpu.VMEM((ECP,), I32),
      pltpu.VMEM((ECP,), I32),
      pltpu.VMEM((EB,), I32),
      pltpu.VMEM((EB,), I32),
      pltpu.VMEM((EB,), I32),
      pltpu.VMEM((EB,), I32),
      pltpu.VMEM((EB, HHALF), F32),
      pltpu.VMEM((EB, HHALF), F32),
      pltpu.SemaphoreType.DMA,
      pltpu.SemaphoreType.DMA,
      pltpu.SemaphoreType.DMA,
      pltpu.SemaphoreType.DMA,
  ]


def _sc_b_body(issue_gather, wait_gather, c, s,
               src_hbm, dst_hbm, scat,
               acc_s, srcb, dstb, idxga, idxgb, idxs, idxs2, rowsa, rowsb,
               sema, semb, semsa, semsb, coff):
  padv = jnp.full((16,), N, I32) + (s % 8)
  for k in range(EC, ECP, 16):
    srcb[pl.ds(k, 16)] = padv
    dstb[pl.ds(k, 16)] = padv
  pltpu.sync_copy(src_hbm.at[pl.ds(s * EC, EC)], srcb.at[pl.ds(0, EC)])
  pltpu.sync_copy(dst_hbm.at[pl.ds(s * EC, EC)], dstb.at[pl.ds(0, EC)])

  z16 = jnp.zeros((16,), F32)
  for r in range(EB):
    for j in range(HHALF // 16):
      rowsa[r, pl.ds(j * 16, 16)] = z16
  off = 0
  while off < NCH:
    step = min(EB, NCH - off)
    pltpu.sync_copy(rowsa.at[pl.ds(0, step), :],
                    acc_s.at[pl.ds(s * NCH + off, step), :])
    off += step
  plsc.subcore_barrier()

  def ed2(k, _):
    i0 = 2 * k
    _copyeb(srcb, i0 * EB, idxga, add=coff)
    issue_gather(idxga, rowsa, sema)
    @pl.when(k > 0)
    def _():
      wait_gather(idxgb, rowsb, semb)
      _copyeb(dstb, (i0 - 1) * EB, idxs)
      pltpu.sync_copy(rowsb, acc_s.at[idxs], add=True)
    _copyeb(srcb, (i0 + 1) * EB, idxgb, add=coff)
    issue_gather(idxgb, rowsb, semb)
    wait_gather(idxga, rowsa, sema)
    _copyeb(dstb, i0 * EB, idxs)
    pltpu.sync_copy(rowsa, acc_s.at[idxs], add=True)
    return 0
  lax.fori_loop(0, NB // 2, ed2, 0)
  wait_gather(idxgb, rowsb, semb)
  _copyeb(dstb, (NB - 1) * EB, idxs)
  pltpu.sync_copy(rowsb, acc_s.at[idxs], add=True)
  plsc.subcore_barrier()

  pltpu.sync_copy(acc_s.at[pl.ds(s * NCH, NCH), :],
                  scat.at[pl.ds(c * NP + s * NCH, NCH), :])


@functools.partial(
    pl.kernel,
    out_type=jax.ShapeDtypeStruct((2 * NP, HHALF), F32),
    mesh=_mesh,
    compiler_params=pltpu.CompilerParams(needs_layout_passes=False),
    scratch_types=_sc_b_scratch(),
)
def _sc_pass_b_cat(src_hbm, dst_hbm, gcat, scat, *scr):
  c = lax.axis_index("c")
  s = lax.axis_index("s")
  coff = jnp.full((16,), 0, I32) + c * NP

  def issue(idxref, rowsref, sem):
    pltpu.async_copy(gcat.at[idxref], rowsref, sem)

  def wait(idxref, rowsref, sem):
    pltpu.make_async_copy(gcat.at[idxref], rowsref, sem).wait()

  _sc_b_body(issue, wait, c, s, src_hbm, dst_hbm, scat, *scr, coff=coff)


@functools.partial(
    pl.kernel,
    out_type=jax.ShapeDtypeStruct((2 * NP, HHALF), F32),
    mesh=_mesh,
    compiler_params=pltpu.CompilerParams(needs_layout_passes=False),
    scratch_types=_sc_b_scratch(),
)
def _sc_pass_b_ab(src_hbm, dst_hbm, ga, gb, scat, *scr):
  c = lax.axis_index("c")
  s = lax.axis_index("s")
  coff = jnp.full((16,), 0, I32)

  def issue(idxref, rowsref, sem):
    @pl.when(c == 0)
    def _():
      pltpu.async_copy(ga.at[idxref], rowsref, sem)
    @pl.when(c == 1)
    def _():
      pltpu.async_copy(gb.at[idxref], rowsref, sem)

  def wait(idxref, rowsref, sem):
    pltpu.make_async_copy(ga.at[idxref], rowsref, sem).wait()

  _sc_b_body(issue, wait, c, s, src_hbm, dst_hbm, scat, *scr, coff=coff)


def _tc_layer_body(dinv_ref, sa_ref, sb_ref, ga_ref, gb_ref, w_ref, b_ref,
                   oa_ref, ob_ref):
  dv = dinv_ref[...]
  t = jnp.concatenate(
      [sa_ref[...] + ga_ref[...], sb_ref[...] + gb_ref[...]], axis=1) * dv
  h = jnp.maximum(
      jnp.dot(t, w_ref[...], preferred_element_type=F32) + b_ref[...], 0.0)
  gn = h * dv
  oa_ref[...] = gn[:, :HHALF]
  ob_ref[...] = gn[:, HHALF:]


_tc_layer = pl.pallas_call(
    _tc_layer_body,
    grid=(NBLKF,),
    in_specs=[
        pl.BlockSpec((BLKF, 1), lambda i: (i, 0)),
        pl.BlockSpec((BLKF, HHALF), lambda i: (i, 0)),
        pl.BlockSpec((BLKF, HHALF), lambda i: (NBLKF + i, 0)),
        pl.BlockSpec((BLKF, HHALF), lambda i: (i, 0)),
        pl.BlockSpec((BLKF, HHALF), lambda i: (NBLKF + i, 0)),
        pl.BlockSpec((HID, HID), lambda i: (0, 0)),
        pl.BlockSpec((1, HID), lambda i: (0, 0)),
    ],
    out_specs=[
        pl.BlockSpec((BLKF, HHALF), lambda i: (i, 0)),
        pl.BlockSpec((BLKF, HHALF), lambda i: (i, 0)),
    ],
    out_shape=[
        jax.ShapeDtypeStruct((NP, HHALF), F32),
        jax.ShapeDtypeStruct((NP, HHALF), F32),
    ],
)


def _tc_final_body(dinv_ref, sa_ref, sb_ref, ga_ref, gb_ref, w3_ref, b3_ref,
                   batch_ref, nd_ref, m1w_ref, m1b_ref, m2w_ref, m2b_ref,
                   m3w_ref, m3b_ref, fw_ref, fb_ref,
                   out_ref, pool_acc, cnt_acc):
  i = pl.program_id(0)
  dv = dinv_ref[...]
  t = jnp.concatenate(
      [sa_ref[...] + ga_ref[...], sb_ref[...] + gb_ref[...]], axis=1) * dv
  h = jnp.maximum(
      jnp.dot(t, w3_ref[...], preferred_element_type=F32) + b3_ref[...], 0.0)
  onehot = (lax.broadcasted_iota(I32, (G, BLKF), 0)
            == batch_ref[...]).astype(F32)
  psum = jnp.dot(onehot, h, preferred_element_type=F32)
  pcnt = jnp.sum(onehot, axis=1, keepdims=True)

  @pl.when(i == 0)
  def _():
    pool_acc[...] = jnp.zeros_like(pool_acc)
    cnt_acc[...] = jnp.zeros_like(cnt_acc)

  pool_acc[...] += psum
  cnt_acc[...] += pcnt

  @pl.when(i == NBLKF - 1)
  def _():
    emb = pool_acc[...] / jnp.maximum(cnt_acc[...], 1.0)
    z = jnp.maximum(
        jnp.dot(nd_ref[...], m1w_ref[...], preferred_element_type=F32)
        + m1b_ref[...], 0.0)
    z = jnp.maximum(
        jnp.dot(z, m2w_ref[...], preferred_element_type=F32)
        + m2b_ref[...], 0.0)
    z = jnp.dot(z, m3w_ref[...], preferred_element_type=F32) + m3b_ref[...]
    res = (jnp.dot(emb, fw_ref[:HID, :], preferred_element_type=F32)
           + jnp.dot(z, fw_ref[HID:, :], preferred_element_type=F32)
           + fb_ref[...])
    out_ref[...] = res


_tc_final = pl.pallas_call(
    _tc_final_body,
    grid=(NBLKF,),
    in_specs=[
        pl.BlockSpec((BLKF, 1), lambda i: (i, 0)),
        pl.BlockSpec((BLKF, HHALF), lambda i: (i, 0)),
        pl.BlockSpec((BLKF, HHALF), lambda i: (NBLKF + i, 0)),
        pl.BlockSpec((BLKF, HHALF), lambda i: (i, 0)),
        pl.BlockSpec((BLKF, HHALF), lambda i: (i, 0)),
        pl.BlockSpec((HID, HID), lambda i: (0, 0)),
        pl.BlockSpec((1, HID), lambda i: (0, 0)),
        pl.BlockSpec((1, BLKF), lambda i: (0, i)),
        pl.BlockSpec((G, MLP_IN), lambda i: (0, 0)),
        pl.BlockSpec((MLP_IN, HID), lambda i: (0, 0)),
        pl.BlockSpec((1, HID), lambda i: (0, 0)),
        pl.BlockSpec((HID, HID), lambda i: (0, 0)),
        pl.BlockSpec((1, HID), lambda i: (0, 0)),
        pl.BlockSpec((HID, HID), lambda i: (0, 0)),
        pl.BlockSpec((1, HID), lambda i: (0, 0)),
        pl.BlockSpec((HID + HID, OUT), lambda i: (0, 0)),
        pl.BlockSpec((1, OUT), lambda i: (0, 0)),
    ],
    out_specs=pl.BlockSpec((G, OUT), lambda i: (0, 0)),
    out_shape=jax.ShapeDtypeStruct((G, OUT), F32),
    scratch_shapes=[
        pltpu.VMEM((G, HID), F32),
        pltpu.VMEM((G, 1), F32),
    ],
)


def kernel(x, edge_index, batch, numerical_data, W1, b1, W2, b2, W3, b3,
           M1w, M1b, M2w, M2b, M3w, M3b, Fw, Fb):
  src = edge_index[0]
  dst = edge_index[1]
  xpad = jnp.pad(x[:, 0], (0, NP - N))
  batch2d = jnp.pad(batch, (0, NP - N), constant_values=G).reshape(1, NP)

  gcat, dinv = _sc_pass_a(src, dst, xpad, W1.reshape(HID), b1)
  dinv2d = dinv.reshape(NP, 1)

  scat = _sc_pass_b_cat(src, dst, gcat)
  g2a, g2b = _tc_layer(dinv2d, scat, scat, gcat, gcat, W2,
                       b2.reshape(1, HID))

  scat3 = _sc_pass_b_ab(src, dst, g2a, g2b)
  out = _tc_final(dinv2d, scat3, scat3, g2a, g2b, W3, b3.reshape(1, HID),
                  batch2d, numerical_data,
                  M1w, M1b.reshape(1, HID), M2w, M2b.reshape(1, HID),
                  M3w, M3b.reshape(1, HID), Fw, Fb.reshape(1, OUT))
  return out


# speedup vs baseline: 1.2758x; 1.0031x over previous
import functools

import jax
import jax.numpy as jnp
from jax import lax
from jax.experimental import pallas as pl
from jax.experimental.pallas import tpu as pltpu
from jax.experimental.pallas import tpu_sc as plsc

N = 10000
NP = 10240
E = 160000
G = 512
HID = 256
HHALF = 128
MLP_IN = 128
OUT = 128

NC = 2
NS = 16
EC = E // NS
EB = 96
NB = 106
EBS = 128
NBS = 80
ECP = NBS * EBS
ECPB = NB * EB
NCH = NP // NS
BLK = 256
NBLK = NP // BLK
BLKF = 512
NBLKF = NP // BLKF

_mesh = plsc.VectorSubcoreMesh(
    core_axis_name="c", subcore_axis_name="s", num_cores=NC, num_subcores=NS)

F32 = jnp.float32
I32 = jnp.int32


def _rsqrt16(d):
  i = plsc.bitcast(d, I32)
  y = plsc.bitcast(jnp.int32(0x5F3759DF) - (i >> 1), F32)
  for _ in range(3):
    y = y * (1.5 - 0.5 * d * y * y)
  return y


def _copyn(n, src_ref, off, dst_ref, add=None):
  for j in range(n // 16):
    v = src_ref[pl.ds(off + j * 16, 16)]
    if add is not None:
      v = v + add
    dst_ref[pl.ds(j * 16, 16)] = v


def _copyeb(src_ref, off, dst_ref, add=None):
  _copyn(EB, src_ref, off, dst_ref, add=add)


@functools.partial(
    pl.kernel,
    out_type=(
        jax.ShapeDtypeStruct((2 * NP, HHALF), F32),
        jax.ShapeDtypeStruct((NP,), F32),
    ),
    mesh=_mesh,
    compiler_params=pltpu.CompilerParams(needs_layout_passes=False),
    scratch_types=[
        pltpu.VMEM_SHARED((NP,), F32),
        pltpu.VMEM_SHARED((NP,), F32),
        pltpu.VMEM_SHARED((NP,), F32),
        pltpu.VMEM((ECP,), I32),
        pltpu.VMEM((ECP,), I32),
        pltpu.VMEM((EBS,), I32),
        pltpu.VMEM((EBS,), I32),
        pltpu.VMEM((EBS,), F32),
        pltpu.VMEM((EBS,), I32),
        pltpu.VMEM((EBS,), I32),
        pltpu.VMEM((EBS,), F32),
        pltpu.SemaphoreType.DMA,
        pltpu.SemaphoreType.DMA,
        pltpu.VMEM((NCH,), F32),
        pltpu.VMEM((NCH,), F32),
        pltpu.VMEM((NCH,), F32),
        pltpu.VMEM((NCH,), F32),
        pltpu.VMEM((NCH,), F32),
        pltpu.VMEM((HHALF,), F32),
        pltpu.VMEM((HHALF,), F32),
        pltpu.VMEM((32, HHALF), F32),
    ],
)
def _sc_pass_a(src_hbm, dst_hbm, x_hbm, w1_hbm, b1_hbm,
               gcat, dinv_out,
               deg_s, g1_s, s1_s,
               srcb, dstb, idxg, idxs, valb, idxg2, idxs2, valb2,
               sema, semb,
               degc, dinvc, xc, g1c, s1c, w1b, b1b, gout):
  c = lax.axis_index("c")
  s = lax.axis_index("s")
  nslice = pl.ds(s * NCH, NCH)

  padv = jnp.full((16,), N, I32) + (s % 8)
  for k in range(EC, ECP, 16):
    srcb[pl.ds(k, 16)] = padv
    dstb[pl.ds(k, 16)] = padv
  pltpu.sync_copy(src_hbm.at[pl.ds(s * EC, EC)], srcb.at[pl.ds(0, EC)])
  pltpu.sync_copy(dst_hbm.at[pl.ds(s * EC, EC)], dstb.at[pl.ds(0, EC)])

  def zf(k, _):
    degc[pl.ds(k * 16, 16)] = jnp.zeros((16,), F32)
    return 0
  lax.fori_loop(0, NCH // 16, zf, 0)
  pltpu.sync_copy(degc, deg_s.at[nslice])
  pltpu.sync_copy(degc, s1_s.at[nslice])
  for k in range(0, EBS, 16):
    valb[pl.ds(k, 16)] = jnp.ones((16,), F32)
    valb2[pl.ds(k, 16)] = jnp.ones((16,), F32)
  plsc.subcore_barrier()

  def deg_it(k, _):
    @pl.when(k > 0)
    def _():
      pltpu.make_async_copy(valb, deg_s.at[idxs], sema).wait()
    _copyn(EBS, dstb, (2 * k) * EBS, idxs)
    pltpu.async_copy(valb, deg_s.at[idxs], sema, add=True)
    @pl.when(k > 0)
    def _():
      pltpu.make_async_copy(valb2, deg_s.at[idxs2], semb).wait()
    _copyn(EBS, dstb, (2 * k + 1) * EBS, idxs2)
    pltpu.async_copy(valb2, deg_s.at[idxs2], semb, add=True)
    return 0
  lax.fori_loop(0, NBS // 2, deg_it, 0)
  pltpu.make_async_copy(valb, deg_s.at[idxs], sema).wait()
  pltpu.make_async_copy(valb2, deg_s.at[idxs2], semb).wait()
  plsc.subcore_barrier()

  pltpu.sync_copy(deg_s.at[nslice], degc)
  pltpu.sync_copy(x_hbm.at[nslice], xc)
  def dv(k, _):
    ks = pl.ds(k * 16, 16)
    y = _rsqrt16(degc[ks] + 1.0)
    dinvc[ks] = y
    g1c[ks] = y * xc[ks]
    return 0
  lax.fori_loop(0, NCH // 16, dv, 0)
  pltpu.sync_copy(g1c, g1_s.at[nslice])
  @pl.when(c == 0)
  def _():
    pltpu.sync_copy(dinvc, dinv_out.at[nslice])
  plsc.subcore_barrier()

  def s1_it(k, _):
    i0 = 2 * k
    @pl.when(k > 0)
    def _():
      pltpu.make_async_copy(valb, s1_s.at[idxs], sema).wait()
    _copyn(EBS, srcb, i0 * EBS, idxg)
    pltpu.async_copy(g1_s.at[idxg], valb, sema)
    @pl.when(k > 0)
    def _():
      pltpu.make_async_copy(valb2, s1_s.at[idxs2], semb).wait()
    _copyn(EBS, srcb, (i0 + 1) * EBS, idxg2)
    pltpu.async_copy(g1_s.at[idxg2], valb2, semb)
    pltpu.make_async_copy(g1_s.at[idxg], valb, sema).wait()
    _copyn(EBS, dstb, i0 * EBS, idxs)
    pltpu.async_copy(valb, s1_s.at[idxs], sema, add=True)
    pltpu.make_async_copy(g1_s.at[idxg2], valb2, semb).wait()
    _copyn(EBS, dstb, (i0 + 1) * EBS, idxs2)
    pltpu.async_copy(valb2, s1_s.at[idxs2], semb, add=True)
    return 0
  lax.fori_loop(0, NBS // 2, s1_it, 0)
  pltpu.make_async_copy(valb, s1_s.at[idxs], sema).wait()
  pltpu.make_async_copy(valb2, s1_s.at[idxs2], semb).wait()
  plsc.subcore_barrier()

  pltpu.sync_copy(s1_s.at[nslice], s1c)
  pltpu.sync_copy(w1_hbm.at[pl.ds(c * HHALF, HHALF)], w1b)
  pltpu.sync_copy(b1_hbm.at[pl.ds(c * HHALF, HHALF)], b1b)
  def tv(k, _):
    ks = pl.ds(k * 16, 16)
    s1c[ks] = dinvc[ks] * (s1c[ks] + g1c[ks])
    return 0
  lax.fori_loop(0, NCH // 16, tv, 0)
  w1v = [w1b[pl.ds(j * 16, 16)] for j in range(HHALF // 16)]
  b1v = [b1b[pl.ds(j * 16, 16)] for j in range(HHALF // 16)]
  NH8 = 32
  def gchunk(half, _):
    def nv(kb, _):
      tvec = s1c[pl.ds(half * NH8 + kb * 16, 16)]
      dvec = dinvc[pl.ds(half * NH8 + kb * 16, 16)]
      for l in range(16):
        tb = jnp.full((16,), tvec[l], F32)
        db = jnp.full((16,), dvec[l], F32)
        rn = kb * 16 + l
        for j in range(HHALF // 16):
          gout[rn, pl.ds(j * 16, 16)] = (
              jnp.maximum(tb * w1v[j] + b1v[j], 0.0) * db)
      return 0
    lax.fori_loop(0, NH8 // 16, nv, 0)
    pltpu.sync_copy(
        gout, gcat.at[pl.ds(c * NP + s * NCH + half * NH8, NH8), :])
    return 0
  lax.fori_loop(0, NCH // NH8, gchunk, 0)


def _sc_b_scratch():
  return [
      pltpu.VMEM_SHARED((NP, HHALF), F32),
      pltpu.VMEM((ECPB,), I32),
      pltpu.VMEM((ECPB,), I32),
      pltpu.VMEM((EB,), I32),
      pltpu.VMEM((EB,), I32),
      pltpu.VMEM((EB,), I32),
      pltpu.VMEM((EB, HHALF), F32),
      pltpu.VMEM((EB, HHALF), F32),
      pltpu.SemaphoreType.DMA,
      pltpu.SemaphoreType.DMA,
  ]


def _sc_b_body(issue_gather, wait_gather, c, s,
               src_hbm, dst_hbm, scat,
               acc_s, srcb, dstb, idxga, idxgb, idxs, rowsa, rowsb,
               sema, semb, coff):
  padv = jnp.full((16,), N, I32) + (s % 8)
  for k in range(EC, ECPB, 16):
    srcb[pl.ds(k, 16)] = padv
    dstb[pl.ds(k, 16)] = padv
  pltpu.sync_copy(src_hbm.at[pl.ds(s * EC, EC)], srcb.at[pl.ds(0, EC)])
  pltpu.sync_copy(dst_hbm.at[pl.ds(s * EC, EC)], dstb.at[pl.ds(0, EC)])

  z16 = jnp.zeros((16,), F32)
  for r in range(EB):
    for j in range(HHALF // 16):
      rowsa[r, pl.ds(j * 16, 16)] = z16
  off = 0
  while off < NCH:
    step = min(EB, NCH - off)
    pltpu.sync_copy(rowsa.at[pl.ds(0, step), :],
                    acc_s.at[pl.ds(s * NCH + off, step), :])
    off += step
  plsc.subcore_barrier()

  def ed2(k, _):
    i0 = 2 * k
    _copyeb(srcb, i0 * EB, idxga, add=coff)
    issue_gather(idxga, rowsa, sema)
    @pl.when(k > 0)
    def _():
      wait_gather(idxgb, rowsb, semb)
      _copyeb(dstb, (i0 - 1) * EB, idxs)
      pltpu.sync_copy(rowsb, acc_s.at[idxs], add=True)
    _copyeb(srcb, (i0 + 1) * EB, idxgb, add=coff)
    issue_gather(idxgb, rowsb, semb)
    wait_gather(idxga, rowsa, sema)
    _copyeb(dstb, i0 * EB, idxs)
    pltpu.sync_copy(rowsa, acc_s.at[idxs], add=True)
    return 0
  lax.fori_loop(0, NB // 2, ed2, 0)
  wait_gather(idxgb, rowsb, semb)
  _copyeb(dstb, (NB - 1) * EB, idxs)
  pltpu.sync_copy(rowsb, acc_s.at[idxs], add=True)
  plsc.subcore_barrier()

  pltpu.sync_copy(acc_s.at[pl.ds(s * NCH, NCH), :],
                  scat.at[pl.ds(c * NP + s * NCH, NCH), :])


@functools.partial(
    pl.kernel,
    out_type=jax.ShapeDtypeStruct((2 * NP, HHALF), F32),
    mesh=_mesh,
    compiler_params=pltpu.CompilerParams(needs_layout_passes=False),
    scratch_types=_sc_b_scratch(),
)
def _sc_pass_b_cat(src_hbm, dst_hbm, gcat, scat, *scr):
  c = lax.axis_index("c")
  s = lax.axis_index("s")
  coff = jnp.full((16,), 0, I32) + c * NP

  def issue(idxref, rowsref, sem):
    pltpu.async_copy(gcat.at[idxref], rowsref, sem)

  def wait(idxref, rowsref, sem):
    pltpu.make_async_copy(gcat.at[idxref], rowsref, sem).wait()

  _sc_b_body(issue, wait, c, s, src_hbm, dst_hbm, scat, *scr, coff=coff)


@functools.partial(
    pl.kernel,
    out_type=jax.ShapeDtypeStruct((2 * NP, HHALF), F32),
    mesh=_mesh,
    compiler_params=pltpu.CompilerParams(needs_layout_passes=False),
    scratch_types=_sc_b_scratch(),
)
def _sc_pass_b_ab(src_hbm, dst_hbm, ga, gb, scat, *scr):
  c = lax.axis_index("c")
  s = lax.axis_index("s")
  coff = jnp.full((16,), 0, I32)

  def issue(idxref, rowsref, sem):
    @pl.when(c == 0)
    def _():
      pltpu.async_copy(ga.at[idxref], rowsref, sem)
    @pl.when(c == 1)
    def _():
      pltpu.async_copy(gb.at[idxref], rowsref, sem)

  def wait(idxref, rowsref, sem):
    pltpu.make_async_copy(ga.at[idxref], rowsref, sem).wait()

  _sc_b_body(issue, wait, c, s, src_hbm, dst_hbm, scat, *scr, coff=coff)


def _tc_layer_body(dinv_ref, sa_ref, sb_ref, ga_ref, gb_ref, w_ref, b_ref,
                   oa_ref, ob_ref):
  dv = dinv_ref[...]
  t = jnp.concatenate(
      [sa_ref[...] + ga_ref[...], sb_ref[...] + gb_ref[...]], axis=1) * dv
  h = jnp.maximum(
      jnp.dot(t, w_ref[...], preferred_element_type=F32) + b_ref[...], 0.0)
  gn = h * dv
  oa_ref[...] = gn[:, :HHALF]
  ob_ref[...] = gn[:, HHALF:]


_tc_layer = pl.pallas_call(
    _tc_layer_body,
    grid=(NBLKF,),
    in_specs=[
        pl.BlockSpec((BLKF, 1), lambda i: (i, 0)),
        pl.BlockSpec((BLKF, HHALF), lambda i: (i, 0)),
        pl.BlockSpec((BLKF, HHALF), lambda i: (NBLKF + i, 0)),
        pl.BlockSpec((BLKF, HHALF), lambda i: (i, 0)),
        pl.BlockSpec((BLKF, HHALF), lambda i: (NBLKF + i, 0)),
        pl.BlockSpec((HID, HID), lambda i: (0, 0)),
        pl.BlockSpec((1, HID), lambda i: (0, 0)),
    ],
    out_specs=[
        pl.BlockSpec((BLKF, HHALF), lambda i: (i, 0)),
        pl.BlockSpec((BLKF, HHALF), lambda i: (i, 0)),
    ],
    out_shape=[
        jax.ShapeDtypeStruct((NP, HHALF), F32),
        jax.ShapeDtypeStruct((NP, HHALF), F32),
    ],
)


def _tc_final_body(dinv_ref, sa_ref, sb_ref, ga_ref, gb_ref, w3_ref, b3_ref,
                   batch_ref, nd_ref, m1w_ref, m1b_ref, m2w_ref, m2b_ref,
                   m3w_ref, m3b_ref, fw_ref, fb_ref,
                   out_ref, pool_acc, cnt_acc):
  i = pl.program_id(0)
  dv = dinv_ref[...]
  t = jnp.concatenate(
      [sa_ref[...] + ga_ref[...], sb_ref[...] + gb_ref[...]], axis=1) * dv
  h = jnp.maximum(
      jnp.dot(t, w3_ref[...], preferred_element_type=F32) + b3_ref[...], 0.0)
  onehot = (lax.broadcasted_iota(I32, (G, BLKF), 0)
            == batch_ref[...]).astype(F32)
  psum = jnp.dot(onehot, h, preferred_element_type=F32)
  pcnt = jnp.sum(onehot, axis=1, keepdims=True)

  @pl.when(i == 0)
  def _():
    pool_acc[...] = jnp.zeros_like(pool_acc)
    cnt_acc[...] = jnp.zeros_like(cnt_acc)

  pool_acc[...] += psum
  cnt_acc[...] += pcnt

  @pl.when(i == NBLKF - 1)
  def _():
    emb = pool_acc[...] / jnp.maximum(cnt_acc[...], 1.0)
    z = jnp.maximum(
        jnp.dot(nd_ref[...], m1w_ref[...], preferred_element_type=F32)
        + m1b_ref[...], 0.0)
    z = jnp.maximum(
        jnp.dot(z, m2w_ref[...], preferred_element_type=F32)
        + m2b_ref[...], 0.0)
    z = jnp.dot(z, m3w_ref[...], preferred_element_type=F32) + m3b_ref[...]
    res = (jnp.dot(emb, fw_ref[:HID, :], preferred_element_type=F32)
           + jnp.dot(z, fw_ref[HID:, :], preferred_element_type=F32)
           + fb_ref[...])
    out_ref[...] = res


_tc_final = pl.pallas_call(
    _tc_final_body,
    grid=(NBLKF,),
    in_specs=[
        pl.BlockSpec((BLKF, 1), lambda i: (i, 0)),
        pl.BlockSpec((BLKF, HHALF), lambda i: (i, 0)),
        pl.BlockSpec((BLKF, HHALF), lambda i: (NBLKF + i, 0)),
        pl.BlockSpec((BLKF, HHALF), lambda i: (i, 0)),
        pl.BlockSpec((BLKF, HHALF), lambda i: (i, 0)),
        pl.BlockSpec((HID, HID), lambda i: (0, 0)),
        pl.BlockSpec((1, HID), lambda i: (0, 0)),
        pl.BlockSpec((1, BLKF), lambda i: (0, i)),
        pl.BlockSpec((G, MLP_IN), lambda i: (0, 0)),
        pl.BlockSpec((MLP_IN, HID), lambda i: (0, 0)),
        pl.BlockSpec((1, HID), lambda i: (0, 0)),
        pl.BlockSpec((HID, HID), lambda i: (0, 0)),
        pl.BlockSpec((1, HID), lambda i: (0, 0)),
        pl.BlockSpec((HID, HID), lambda i: (0, 0)),
        pl.BlockSpec((1, HID), lambda i: (0, 0)),
        pl.BlockSpec((HID + HID, OUT), lambda i: (0, 0)),
        pl.BlockSpec((1, OUT), lambda i: (0, 0)),
    ],
    out_specs=pl.BlockSpec((G, OUT), lambda i: (0, 0)),
    out_shape=jax.ShapeDtypeStruct((G, OUT), F32),
    scratch_shapes=[
        pltpu.VMEM((G, HID), F32),
        pltpu.VMEM((G, 1), F32),
    ],
)


def kernel(x, edge_index, batch, numerical_data, W1, b1, W2, b2, W3, b3,
           M1w, M1b, M2w, M2b, M3w, M3b, Fw, Fb):
  src = edge_index[0]
  dst = edge_index[1]
  xpad = jnp.pad(x[:, 0], (0, NP - N))
  batch2d = jnp.pad(batch, (0, NP - N), constant_values=G).reshape(1, NP)

  gcat, dinv = _sc_pass_a(src, dst, xpad, W1.reshape(HID), b1)
  dinv2d = dinv.reshape(NP, 1)

  scat = _sc_pass_b_cat(src, dst, gcat)
  g2a, g2b = _tc_layer(dinv2d, scat, scat, gcat, gcat, W2,
                       b2.reshape(1, HID))

  scat3 = _sc_pass_b_ab(src, dst, g2a, g2b)
  out = _tc_final(dinv2d, scat3, scat3, g2a, g2b, W3, b3.reshape(1, HID),
                  batch2d, numerical_data,
                  M1w, M1b.reshape(1, HID), M2w, M2b.reshape(1, HID),
                  M3w, M3b.reshape(1, HID), Fw, Fb.reshape(1, OUT))
  return out


# speedup vs baseline: 1.2770x; 1.0010x over previous
import functools

import jax
import jax.numpy as jnp
from jax import lax
from jax.experimental import pallas as pl
from jax.experimental.pallas import tpu as pltpu
from jax.experimental.pallas import tpu_sc as plsc

N = 10000
NP = 10240
E = 160000
G = 512
HID = 256
HHALF = 128
MLP_IN = 128
OUT = 128

NC = 2
NS = 16
EC = E // NS
EB = 96
NB = 106
EBS = 128
NBS = 80
ECP = NBS * EBS
ECPB = NB * EB
NCH = NP // NS
BLK = 256
NBLK = NP // BLK
BLKF = 512
NBLKF = NP // BLKF

_mesh = plsc.VectorSubcoreMesh(
    core_axis_name="c", subcore_axis_name="s", num_cores=NC, num_subcores=NS)

F32 = jnp.float32
I32 = jnp.int32


def _rsqrt16(d):
  i = plsc.bitcast(d, I32)
  y = plsc.bitcast(jnp.int32(0x5F3759DF) - (i >> 1), F32)
  for _ in range(3):
    y = y * (1.5 - 0.5 * d * y * y)
  return y


def _copyn(n, src_ref, off, dst_ref, add=None):
  for j in range(n // 16):
    v = src_ref[pl.ds(off + j * 16, 16)]
    if add is not None:
      v = v + add
    dst_ref[pl.ds(j * 16, 16)] = v


def _copyeb(src_ref, off, dst_ref, add=None):
  _copyn(EB, src_ref, off, dst_ref, add=add)


@functools.partial(
    pl.kernel,
    out_type=(
        jax.ShapeDtypeStruct((2 * NP, HHALF), F32),
        jax.ShapeDtypeStruct((NP,), F32),
    ),
    mesh=_mesh,
    compiler_params=pltpu.CompilerParams(needs_layout_passes=False),
    scratch_types=[
        pltpu.VMEM_SHARED((NP,), F32),
        pltpu.VMEM_SHARED((NP,), F32),
        pltpu.VMEM_SHARED((NP,), F32),
        pltpu.VMEM((ECP,), I32),
        pltpu.VMEM((ECP,), I32),
        pltpu.VMEM((EBS,), I32),
        pltpu.VMEM((EBS,), I32),
        pltpu.VMEM((EBS,), F32),
        pltpu.VMEM((EBS,), I32),
        pltpu.VMEM((EBS,), I32),
        pltpu.VMEM((EBS,), F32),
        pltpu.SemaphoreType.DMA,
        pltpu.SemaphoreType.DMA,
        pltpu.VMEM((NCH,), F32),
        pltpu.VMEM((NCH,), F32),
        pltpu.VMEM((NCH,), F32),
        pltpu.VMEM((NCH,), F32),
        pltpu.VMEM((NCH,), F32),
        pltpu.VMEM((HHALF,), F32),
        pltpu.VMEM((HHALF,), F32),
        pltpu.VMEM((32, HHALF), F32),
    ],
)
def _sc_pass_a(src_hbm, dst_hbm, x_hbm, w1_hbm, b1_hbm,
               gcat, dinv_out,
               deg_s, g1_s, s1_s,
               srcb, dstb, idxg, idxs, valb, idxg2, idxs2, valb2,
               sema, semb,
               degc, dinvc, xc, g1c, s1c, w1b, b1b, gout):
  c = lax.axis_index("c")
  s = lax.axis_index("s")
  nslice = pl.ds(s * NCH, NCH)

  padv = jnp.full((16,), N, I32) + (s % 8)
  for k in range(EC, ECP, 16):
    srcb[pl.ds(k, 16)] = padv
    dstb[pl.ds(k, 16)] = padv
  pltpu.sync_copy(src_hbm.at[pl.ds(s * EC, EC)], srcb.at[pl.ds(0, EC)])
  pltpu.sync_copy(dst_hbm.at[pl.ds(s * EC, EC)], dstb.at[pl.ds(0, EC)])

  def zf(k, _):
    degc[pl.ds(k * 16, 16)] = jnp.zeros((16,), F32)
    return 0
  lax.fori_loop(0, NCH // 16, zf, 0)
  pltpu.sync_copy(degc, deg_s.at[nslice])
  pltpu.sync_copy(degc, s1_s.at[nslice])
  for k in range(0, EBS, 16):
    valb[pl.ds(k, 16)] = jnp.ones((16,), F32)
    valb2[pl.ds(k, 16)] = jnp.ones((16,), F32)
  plsc.subcore_barrier()

  def deg_it(k, _):
    @pl.when(k > 0)
    def _():
      pltpu.make_async_copy(valb, deg_s.at[idxs], sema).wait()
    _copyn(EBS, dstb, (2 * k) * EBS, idxs)
    pltpu.async_copy(valb, deg_s.at[idxs], sema, add=True)
    @pl.when(k > 0)
    def _():
      pltpu.make_async_copy(valb2, deg_s.at[idxs2], semb).wait()
    _copyn(EBS, dstb, (2 * k + 1) * EBS, idxs2)
    pltpu.async_copy(valb2, deg_s.at[idxs2], semb, add=True)
    return 0
  lax.fori_loop(0, NBS // 2, deg_it, 0)
  pltpu.make_async_copy(valb, deg_s.at[idxs], sema).wait()
  pltpu.make_async_copy(valb2, deg_s.at[idxs2], semb).wait()
  plsc.subcore_barrier()

  pltpu.sync_copy(deg_s.at[nslice], degc)
  pltpu.sync_copy(x_hbm.at[nslice], xc)
  def dv(k, _):
    ks = pl.ds(k * 16, 16)
    y = _rsqrt16(degc[ks] + 1.0)
    dinvc[ks] = y
    g1c[ks] = y * xc[ks]
    return 0
  lax.fori_loop(0, NCH // 16, dv, 0)
  pltpu.sync_copy(g1c, g1_s.at[nslice])
  @pl.when(c == 0)
  def _():
    pltpu.sync_copy(dinvc, dinv_out.at[nslice])
  plsc.subcore_barrier()

  def s1_it(k, _):
    i0 = 2 * k
    @pl.when(k > 0)
    def _():
      pltpu.make_async_copy(valb, s1_s.at[idxs], sema).wait()
    _copyn(EBS, srcb, i0 * EBS, idxg)
    pltpu.async_copy(g1_s.at[idxg], valb, sema)
    @pl.when(k > 0)
    def _():
      pltpu.make_async_copy(valb2, s1_s.at[idxs2], semb).wait()
    _copyn(EBS, srcb, (i0 + 1) * EBS, idxg2)
    pltpu.async_copy(g1_s.at[idxg2], valb2, semb)
    pltpu.make_async_copy(g1_s.at[idxg], valb, sema).wait()
    _copyn(EBS, dstb, i0 * EBS, idxs)
    pltpu.async_copy(valb, s1_s.at[idxs], sema, add=True)
    pltpu.make_async_copy(g1_s.at[idxg2], valb2, semb).wait()
    _copyn(EBS, dstb, (i0 + 1) * EBS, idxs2)
    pltpu.async_copy(valb2, s1_s.at[idxs2], semb, add=True)
    return 0
  lax.fori_loop(0, NBS // 2, s1_it, 0)
  pltpu.make_async_copy(valb, s1_s.at[idxs], sema).wait()
  pltpu.make_async_copy(valb2, s1_s.at[idxs2], semb).wait()
  plsc.subcore_barrier()

  pltpu.sync_copy(s1_s.at[nslice], s1c)
  pltpu.sync_copy(w1_hbm.at[pl.ds(c * HHALF, HHALF)], w1b)
  pltpu.sync_copy(b1_hbm.at[pl.ds(c * HHALF, HHALF)], b1b)
  def tv(k, _):
    ks = pl.ds(k * 16, 16)
    s1c[ks] = dinvc[ks] * (s1c[ks] + g1c[ks])
    return 0
  lax.fori_loop(0, NCH // 16, tv, 0)
  w1v = [w1b[pl.ds(j * 16, 16)] for j in range(HHALF // 16)]
  b1v = [b1b[pl.ds(j * 16, 16)] for j in range(HHALF // 16)]
  NH8 = 32
  def gchunk(half, _):
    def nv(kb, _):
      tvec = s1c[pl.ds(half * NH8 + kb * 16, 16)]
      dvec = dinvc[pl.ds(half * NH8 + kb * 16, 16)]
      for l in range(16):
        tb = jnp.full((16,), tvec[l], F32)
        db = jnp.full((16,), dvec[l], F32)
        rn = kb * 16 + l
        for j in range(HHALF // 16):
          gout[rn, pl.ds(j * 16, 16)] = (
              jnp.maximum(tb * w1v[j] + b1v[j], 0.0) * db)
      return 0
    lax.fori_loop(0, NH8 // 16, nv, 0)
    pltpu.sync_copy(
        gout, gcat.at[pl.ds(c * NP + s * NCH + half * NH8, NH8), :])
    return 0
  lax.fori_loop(0, NCH // NH8, gchunk, 0)


def _sc_b_scratch():
  return [
      pltpu.VMEM_SHARED((NP, HHALF), F32),
      pltpu.VMEM((ECPB,), I32),
      pltpu.VMEM((ECPB,), I32),
      pltpu.VMEM((EB,), I32),
      pltpu.VMEM((EB,), I32),
      pltpu.VMEM((EB,), I32),
      pltpu.VMEM((EB, HHALF), F32),
      pltpu.VMEM((EB, HHALF), F32),
      pltpu.SemaphoreType.DMA,
      pltpu.SemaphoreType.DMA,
  ]


def _sc_b_body(issue_gather, wait_gather, c, s,
               src_hbm, dst_hbm, scat,
               acc_s, srcb, dstb, idxga, idxgb, idxs, rowsa, rowsb,
               sema, semb, coff):
  padv = jnp.full((16,), N, I32) + (s % 8)
  for k in range(EC, ECPB, 16):
    srcb[pl.ds(k, 16)] = padv
    dstb[pl.ds(k, 16)] = padv
  pltpu.sync_copy(src_hbm.at[pl.ds(s * EC, EC)], srcb.at[pl.ds(0, EC)])
  pltpu.sync_copy(dst_hbm.at[pl.ds(s * EC, EC)], dstb.at[pl.ds(0, EC)])

  z16 = jnp.zeros((16,), F32)
  for r in range(EB):
    for j in range(HHALF // 16):
      rowsa[r, pl.ds(j * 16, 16)] = z16
  off = 0
  while off < NCH:
    step = min(EB, NCH - off)
    pltpu.sync_copy(rowsa.at[pl.ds(0, step), :],
                    acc_s.at[pl.ds(s * NCH + off, step), :])
    off += step
  plsc.subcore_barrier()

  def ed2(k, _):
    i0 = 2 * k
    _copyeb(srcb, i0 * EB, idxga, add=coff)
    issue_gather(idxga, rowsa, sema)
    @pl.when(k > 0)
    def _():
      wait_gather(idxgb, rowsb, semb)
      _copyeb(dstb, (i0 - 1) * EB, idxs)
      pltpu.sync_copy(rowsb, acc_s.at[idxs], add=True)
    _copyeb(srcb, (i0 + 1) * EB, idxgb, add=coff)
    issue_gather(idxgb, rowsb, semb)
    wait_gather(idxga, rowsa, sema)
    _copyeb(dstb, i0 * EB, idxs)
    pltpu.sync_copy(rowsa, acc_s.at[idxs], add=True)
    return 0
  lax.fori_loop(0, NB // 2, ed2, 0)
  wait_gather(idxgb, rowsb, semb)
  _copyeb(dstb, (NB - 1) * EB, idxs)
  pltpu.sync_copy(rowsb, acc_s.at[idxs], add=True)
  plsc.subcore_barrier()

  pltpu.sync_copy(acc_s.at[pl.ds(s * NCH, NCH), :],
                  scat.at[pl.ds(c * NP + s * NCH, NCH), :])


@functools.partial(
    pl.kernel,
    out_type=jax.ShapeDtypeStruct((2 * NP, HHALF), F32),
    mesh=_mesh,
    compiler_params=pltpu.CompilerParams(needs_layout_passes=False),
    scratch_types=_sc_b_scratch(),
)
def _sc_pass_b_cat(src_hbm, dst_hbm, gcat, scat, *scr):
  c = lax.axis_index("c")
  s = lax.axis_index("s")
  coff = jnp.full((16,), 0, I32) + c * NP

  def issue(idxref, rowsref, sem):
    pltpu.async_copy(gcat.at[idxref], rowsref, sem)

  def wait(idxref, rowsref, sem):
    pltpu.make_async_copy(gcat.at[idxref], rowsref, sem).wait()

  _sc_b_body(issue, wait, c, s, src_hbm, dst_hbm, scat, *scr, coff=coff)


@functools.partial(
    pl.kernel,
    out_type=jax.ShapeDtypeStruct((2 * NP, HHALF), F32),
    mesh=_mesh,
    compiler_params=pltpu.CompilerParams(needs_layout_passes=False),
    scratch_types=_sc_b_scratch(),
)
def _sc_pass_b_ab(src_hbm, dst_hbm, ga, gb, scat, *scr):
  c = lax.axis_index("c")
  s = lax.axis_index("s")
  coff = jnp.full((16,), 0, I32)

  def issue(idxref, rowsref, sem):
    @pl.when(c == 0)
    def _():
      pltpu.async_copy(ga.at[idxref], rowsref, sem)
    @pl.when(c == 1)
    def _():
      pltpu.async_copy(gb.at[idxref], rowsref, sem)

  def wait(idxref, rowsref, sem):
    pltpu.make_async_copy(ga.at[idxref], rowsref, sem).wait()

  _sc_b_body(issue, wait, c, s, src_hbm, dst_hbm, scat, *scr, coff=coff)


def _tc_layer_body(dinv_ref, sa_ref, sb_ref, ga_ref, gb_ref, w_ref, b_ref,
                   oa_ref, ob_ref):
  dv = dinv_ref[...]
  t = jnp.concatenate(
      [sa_ref[...] + ga_ref[...], sb_ref[...] + gb_ref[...]], axis=1) * dv
  h = jnp.maximum(
      jnp.dot(t, w_ref[...], preferred_element_type=F32) + b_ref[...], 0.0)
  gn = h * dv
  oa_ref[...] = gn[:, :HHALF]
  ob_ref[...] = gn[:, HHALF:]


_tc_layer = pl.pallas_call(
    _tc_layer_body,
    grid=(NBLKF,),
    in_specs=[
        pl.BlockSpec((BLKF, 1), lambda i: (i, 0)),
        pl.BlockSpec((BLKF, HHALF), lambda i: (i, 0)),
        pl.BlockSpec((BLKF, HHALF), lambda i: (NBLKF + i, 0)),
        pl.BlockSpec((BLKF, HHALF), lambda i: (i, 0)),
        pl.BlockSpec((BLKF, HHALF), lambda i: (NBLKF + i, 0)),
        pl.BlockSpec((HID, HID), lambda i: (0, 0)),
        pl.BlockSpec((1, HID), lambda i: (0, 0)),
    ],
    out_specs=[
        pl.BlockSpec((BLKF, HHALF), lambda i: (i, 0)),
        pl.BlockSpec((BLKF, HHALF), lambda i: (i, 0)),
    ],
    out_shape=[
        jax.ShapeDtypeStruct((NP, HHALF), F32),
        jax.ShapeDtypeStruct((NP, HHALF), F32),
    ],
)


def _tc_final_body(dinv_ref, sa_ref, sb_ref, ga_ref, gb_ref, w3_ref, b3_ref,
                   batch_ref, nd_ref, m1w_ref, m1b_ref, m2w_ref, m2b_ref,
                   m3w_ref, m3b_ref, fw_ref, fb_ref,
                   out_ref, pool_acc, cnt_acc):
  i = pl.program_id(0)
  dv = dinv_ref[...]
  t = jnp.concatenate(
      [sa_ref[...] + ga_ref[...], sb_ref[...] + gb_ref[...]], axis=1) * dv
  h = jnp.maximum(
      jnp.dot(t, w3_ref[...], preferred_element_type=F32) + b3_ref[...], 0.0)
  onehot = (lax.broadcasted_iota(I32, (G, BLKF), 0)
            == batch_ref[...]).astype(jnp.bfloat16)
  psum = jnp.dot(onehot, h, preferred_element_type=F32)
  pcnt = jnp.sum(onehot.astype(F32), axis=1, keepdims=True)

  @pl.when(i == 0)
  def _():
    pool_acc[...] = jnp.zeros_like(pool_acc)
    cnt_acc[...] = jnp.zeros_like(cnt_acc)

  pool_acc[...] += psum
  cnt_acc[...] += pcnt

  @pl.when(i == NBLKF - 1)
  def _():
    emb = pool_acc[...] / jnp.maximum(cnt_acc[...], 1.0)
    z = jnp.maximum(
        jnp.dot(nd_ref[...], m1w_ref[...], preferred_element_type=F32)
        + m1b_ref[...], 0.0)
    z = jnp.maximum(
        jnp.dot(z, m2w_ref[...], preferred_element_type=F32)
        + m2b_ref[...], 0.0)
    z = jnp.dot(z, m3w_ref[...], preferred_element_type=F32) + m3b_ref[...]
    res = (jnp.dot(emb, fw_ref[:HID, :], preferred_element_type=F32)
           + jnp.dot(z, fw_ref[HID:, :], preferred_element_type=F32)
           + fb_ref[...])
    out_ref[...] = res


_tc_final = pl.pallas_call(
    _tc_final_body,
    grid=(NBLKF,),
    in_specs=[
        pl.BlockSpec((BLKF, 1), lambda i: (i, 0)),
        pl.BlockSpec((BLKF, HHALF), lambda i: (i, 0)),
        pl.BlockSpec((BLKF, HHALF), lambda i: (NBLKF + i, 0)),
        pl.BlockSpec((BLKF, HHALF), lambda i: (i, 0)),
        pl.BlockSpec((BLKF, HHALF), lambda i: (i, 0)),
        pl.BlockSpec((HID, HID), lambda i: (0, 0)),
        pl.BlockSpec((1, HID), lambda i: (0, 0)),
        pl.BlockSpec((1, BLKF), lambda i: (0, i)),
        pl.BlockSpec((G, MLP_IN), lambda i: (0, 0)),
        pl.BlockSpec((MLP_IN, HID), lambda i: (0, 0)),
        pl.BlockSpec((1, HID), lambda i: (0, 0)),
        pl.BlockSpec((HID, HID), lambda i: (0, 0)),
        pl.BlockSpec((1, HID), lambda i: (0, 0)),
        pl.BlockSpec((HID, HID), lambda i: (0, 0)),
        pl.BlockSpec((1, HID), lambda i: (0, 0)),
        pl.BlockSpec((HID + HID, OUT), lambda i: (0, 0)),
        pl.BlockSpec((1, OUT), lambda i: (0, 0)),
    ],
    out_specs=pl.BlockSpec((G, OUT), lambda i: (0, 0)),
    out_shape=jax.ShapeDtypeStruct((G, OUT), F32),
    scratch_shapes=[
        pltpu.VMEM((G, HID), F32),
        pltpu.VMEM((G, 1), F32),
    ],
)


def kernel(x, edge_index, batch, numerical_data, W1, b1, W2, b2, W3, b3,
           M1w, M1b, M2w, M2b, M3w, M3b, Fw, Fb):
  src = edge_index[0]
  dst = edge_index[1]
  xpad = jnp.pad(x[:, 0], (0, NP - N))
  batch2d = jnp.pad(batch, (0, NP - N), constant_values=G).reshape(1, NP)

  gcat, dinv = _sc_pass_a(src, dst, xpad, W1.reshape(HID), b1)
  dinv2d = dinv.reshape(NP, 1)

  scat = _sc_pass_b_cat(src, dst, gcat)
  g2a, g2b = _tc_layer(dinv2d, scat, scat, gcat, gcat, W2,
                       b2.reshape(1, HID))

  scat3 = _sc_pass_b_ab(src, dst, g2a, g2b)
  out = _tc_final(dinv2d, scat3, scat3, g2a, g2b, W3, b3.reshape(1, HID),
                  batch2d, numerical_data,
                  M1w, M1b.reshape(1, HID), M2w, M2b.reshape(1, HID),
                  M3w, M3b.reshape(1, HID), Fw, Fb.reshape(1, OUT))
  return out
